# baseline (speedup 1.0000x reference)
import functools

import jax
import jax.numpy as jnp
from jax import lax
from jax.experimental import pallas as pl
from jax.experimental.pallas import tpu as pltpu
from jax.experimental.pallas import tpu_sc as plsc

_B = 16384
_D = 64

_NC, _NS = 2, 16
_NW = _NC * _NS
_BPW = _B // _NW


@functools.cache
def _make_sc_gather(n_pairs):
    mesh = plsc.VectorSubcoreMesh(core_axis_name="c", subcore_axis_name="s")

    @functools.partial(
        pl.kernel,
        mesh=mesh,
        out_type=jax.ShapeDtypeStruct((_B, 2 * _D), jnp.float32),
        scratch_types=[
            pltpu.VMEM((_BPW,), jnp.int32),
            pltpu.VMEM((_BPW, 2 * _D), jnp.float32),
            pltpu.SemaphoreType.DMA,
        ],
    )
    def _sc_gather(idx_hbm, table_hbm, out_hbm, idx_v, rows_v, sem):
        wid = lax.axis_index("s") * _NC + lax.axis_index("c")
        base = wid * _BPW
        pltpu.sync_copy(idx_hbm.at[pl.ds(base, _BPW)], idx_v)
        pltpu.async_copy(table_hbm.at[idx_v], rows_v, sem).wait()
        pltpu.sync_copy(rows_v, out_hbm.at[pl.ds(base, _BPW)])

    return _sc_gather


def _mlp_body(x_ref, par_ref, w1_ref, b1_ref, w2_ref, b2_ref, o_ref):
    xl = x_ref[:, :_D]
    xr = x_ref[:, _D:]
    x = jnp.where(par_ref[...] > 0, xr, xl)
    h = jnp.dot(x, w1_ref[...], preferred_element_type=jnp.float32) + b1_ref[...]
    h = h * jax.nn.sigmoid(h)
    o_ref[...] = (
        jnp.dot(h, w2_ref[...], preferred_element_type=jnp.float32) + b2_ref[...]
    )


_BLK = 2048


@jax.jit
def _mlp(x, parity, W1, b1, W2, b2):
    grid = (_B // _BLK,)
    return pl.pallas_call(
        _mlp_body,
        grid=grid,
        in_specs=[
            pl.BlockSpec((_BLK, 2 * _D), lambda i: (i, 0)),
            pl.BlockSpec((_BLK, 1), lambda i: (i, 0)),
            pl.BlockSpec((_D, _D), lambda i: (0, 0)),
            pl.BlockSpec((1, _D), lambda i: (0, 0)),
            pl.BlockSpec((_D, _D), lambda i: (0, 0)),
            pl.BlockSpec((1, _D), lambda i: (0, 0)),
        ],
        out_specs=pl.BlockSpec((_BLK, _D), lambda i: (i, 0)),
        out_shape=jax.ShapeDtypeStruct((_B, _D), jnp.float32),
    )(x, parity, W1, b1, W2, b2)


@jax.jit
def kernel(c, emb_table, W1, b1, W2, b2):
    n = emb_table.shape[0]
    table2 = emb_table.reshape(n // 2, 2 * _D)
    pair_idx = lax.shift_right_logical(c, 1)
    parity = (c & 1).astype(jnp.int32).reshape(_B, 1)
    gathered = _make_sc_gather(n // 2)(pair_idx, table2)
    return _mlp(gathered, parity, W1, b1.reshape(1, _D), W2, b2.reshape(1, _D))


# speedup vs baseline: 1.7450x; 1.7450x over previous
import functools

import jax
import jax.numpy as jnp
from jax import lax
from jax.experimental import pallas as pl
from jax.experimental.pallas import tpu as pltpu
from jax.experimental.pallas import tpu_sc as plsc

_B = 16384
_D = 64
_V = 1000000

_G = 2048
_NCH = _V // _G
_NP = (_V + 2 * _G - 1) // (2 * _G)
_ROWS = _NP * _G

_NC, _NS = 2, 16
_NW = _NC * _NS
_BPW = _B // _NW


def _tr_body(a_ref, b_ref, o_ref):
    o_ref[:, :_D] = a_ref[...].T
    o_ref[:, _D:] = b_ref[...].T


@jax.jit
def _transpose_pairs(tt):
    return pl.pallas_call(
        _tr_body,
        grid=(_NP,),
        in_specs=[
            pl.BlockSpec((_D, _G), lambda i: (0, 2 * i)),
            pl.BlockSpec((_D, _G), lambda i: (0, jnp.minimum(2 * i + 1, _NCH))),
        ],
        out_specs=pl.BlockSpec((_G, 2 * _D), lambda i: (i, 0)),
        out_shape=jax.ShapeDtypeStruct((_ROWS, 2 * _D), jnp.float32),
    )(tt, tt)


@functools.cache
def _make_sc_gather():
    mesh = plsc.VectorSubcoreMesh(core_axis_name="c", subcore_axis_name="s")

    @functools.partial(
        pl.kernel,
        mesh=mesh,
        out_type=jax.ShapeDtypeStruct((_B, 2 * _D), jnp.float32),
        scratch_types=[
            pltpu.VMEM((_BPW,), jnp.int32),
            pltpu.VMEM((_BPW, 2 * _D), jnp.float32),
            pltpu.SemaphoreType.DMA,
        ],
    )
    def _sc_gather(idx_hbm, table_hbm, out_hbm, idx_v, rows_v, sem):
        wid = lax.axis_index("s") * _NC + lax.axis_index("c")
        base = wid * _BPW
        pltpu.sync_copy(idx_hbm.at[pl.ds(base, _BPW)], idx_v)
        pltpu.async_copy(table_hbm.at[idx_v], rows_v, sem).wait()
        pltpu.sync_copy(rows_v, out_hbm.at[pl.ds(base, _BPW)])

    return _sc_gather


def _mlp_body(x_ref, par_ref, w1_ref, b1_ref, w2_ref, b2_ref, o_ref):
    xl = x_ref[:, :_D]
    xr = x_ref[:, _D:]
    x = jnp.where(par_ref[...] > 0, xr, xl)
    h = jnp.dot(x, w1_ref[...], preferred_element_type=jnp.float32) + b1_ref[...]
    h = h * jax.nn.sigmoid(h)
    o_ref[...] = (
        jnp.dot(h, w2_ref[...], preferred_element_type=jnp.float32) + b2_ref[...]
    )


_BLK = 2048


@jax.jit
def _mlp(x, parity, W1, b1, W2, b2):
    grid = (_B // _BLK,)
    return pl.pallas_call(
        _mlp_body,
        grid=grid,
        in_specs=[
            pl.BlockSpec((_BLK, 2 * _D), lambda i: (i, 0)),
            pl.BlockSpec((_BLK, 1), lambda i: (i, 0)),
            pl.BlockSpec((_D, _D), lambda i: (0, 0)),
            pl.BlockSpec((1, _D), lambda i: (0, 0)),
            pl.BlockSpec((_D, _D), lambda i: (0, 0)),
            pl.BlockSpec((1, _D), lambda i: (0, 0)),
        ],
        out_specs=pl.BlockSpec((_BLK, _D), lambda i: (i, 0)),
        out_shape=jax.ShapeDtypeStruct((_B, _D), jnp.float32),
    )(x, parity, W1, b1, W2, b2)


@jax.jit
def kernel(c, emb_table, W1, b1, W2, b2):
    trm = _transpose_pairs(emb_table.T)
    q = c // _G
    row = (q // 2) * _G + (c % _G)
    half = (q % 2).astype(jnp.int32).reshape(_B, 1)
    gathered = _make_sc_gather()(row, trm)
    return _mlp(gathered, half, W1, b1.reshape(1, _D), W2, b2.reshape(1, _D))


# speedup vs baseline: 2.3918x; 1.3707x over previous
import functools

import jax
import jax.numpy as jnp
from jax import lax
from jax.experimental import pallas as pl
from jax.experimental.pallas import tpu as pltpu
from jax.experimental.pallas import tpu_sc as plsc

_B = 16384
_D = 64
_V = 1000000

_G = 8192
_NP = (_V + 2 * _G - 1) // (2 * _G)
_ROWS = _NP * _G

_NC, _NS = 2, 16
_NW = _NC * _NS
_BPW = _B // _NW


def _tr_body(x_ref, o_ref):
    y = x_ref[...].T
    o_ref[:, :_D] = y[:_G]
    o_ref[:, _D:] = y[_G:]


@jax.jit
def _transpose_pairs(tt):
    return pl.pallas_call(
        _tr_body,
        grid=(_NP,),
        in_specs=[pl.BlockSpec((_D, 2 * _G), lambda i: (0, i))],
        out_specs=pl.BlockSpec((_G, 2 * _D), lambda i: (i, 0)),
        out_shape=jax.ShapeDtypeStruct((_ROWS, 2 * _D), jnp.float32),
    )(tt)


@functools.cache
def _make_sc_gather():
    mesh = plsc.VectorSubcoreMesh(core_axis_name="c", subcore_axis_name="s")

    @functools.partial(
        pl.kernel,
        mesh=mesh,
        out_type=jax.ShapeDtypeStruct((_B, 2 * _D), jnp.float32),
        scratch_types=[
            pltpu.VMEM((_BPW,), jnp.int32),
            pltpu.VMEM((_BPW, 2 * _D), jnp.float32),
            pltpu.SemaphoreType.DMA,
        ],
    )
    def _sc_gather(idx_hbm, table_hbm, out_hbm, idx_v, rows_v, sem):
        wid = lax.axis_index("s") * _NC + lax.axis_index("c")
        base = wid * _BPW
        pltpu.sync_copy(idx_hbm.at[pl.ds(base, _BPW)], idx_v)
        pltpu.async_copy(table_hbm.at[idx_v], rows_v, sem).wait()
        pltpu.sync_copy(rows_v, out_hbm.at[pl.ds(base, _BPW)])

    return _sc_gather


def _mlp_body(x_ref, par_ref, w1_ref, b1_ref, w2_ref, b2_ref, o_ref):
    xl = x_ref[:, :_D]
    xr = x_ref[:, _D:]
    x = jnp.where(par_ref[...] > 0, xr, xl)
    h = jnp.dot(x, w1_ref[...], preferred_element_type=jnp.float32) + b1_ref[...]
    h = h * jax.nn.sigmoid(h)
    o_ref[...] = (
        jnp.dot(h, w2_ref[...], preferred_element_type=jnp.float32) + b2_ref[...]
    )


_BLK = 2048


@jax.jit
def _mlp(x, parity, W1, b1, W2, b2):
    grid = (_B // _BLK,)
    return pl.pallas_call(
        _mlp_body,
        grid=grid,
        in_specs=[
            pl.BlockSpec((_BLK, 2 * _D), lambda i: (i, 0)),
            pl.BlockSpec((_BLK, 1), lambda i: (i, 0)),
            pl.BlockSpec((_D, _D), lambda i: (0, 0)),
            pl.BlockSpec((1, _D), lambda i: (0, 0)),
            pl.BlockSpec((_D, _D), lambda i: (0, 0)),
            pl.BlockSpec((1, _D), lambda i: (0, 0)),
        ],
        out_specs=pl.BlockSpec((_BLK, _D), lambda i: (i, 0)),
        out_shape=jax.ShapeDtypeStruct((_B, _D), jnp.float32),
    )(x, parity, W1, b1, W2, b2)


@jax.jit
def kernel(c, emb_table, W1, b1, W2, b2):
    trm = _transpose_pairs(emb_table.T)
    q = c // _G
    row = (q // 2) * _G + (c % _G)
    half = (q % 2).astype(jnp.int32).reshape(_B, 1)
    gathered = _make_sc_gather()(row, trm)
    return _mlp(gathered, half, W1, b1.reshape(1, _D), W2, b2.reshape(1, _D))


# speedup vs baseline: 2.8463x; 1.1900x over previous
import functools

import jax
import jax.numpy as jnp
from jax import lax
from jax.experimental import pallas as pl
from jax.experimental.pallas import tpu as pltpu
from jax.experimental.pallas import tpu_sc as plsc

_B = 16384
_D = 64
_V = 1000000

_G = 4096
_C = 4 * _G
_NP = (_V + _C - 1) // _C
_ROWS = _NP * _G

_NC, _NS = 2, 16
_NW = _NC * _NS
_BPW = _B // _NW


def _to_bf16_bits(v):
    b = lax.bitcast_convert_type(v, jnp.int32)
    return lax.shift_right_logical(b + 0x8000, 16)


def _tr_body(x_ref, o_ref):
    y = x_ref[...].T
    q0 = _to_bf16_bits(y[:_G])
    q1 = _to_bf16_bits(y[_G : 2 * _G])
    q2 = _to_bf16_bits(y[2 * _G : 3 * _G])
    q3 = _to_bf16_bits(y[3 * _G :])
    w01 = lax.shift_left(q0, 16) | q1
    w23 = lax.shift_left(q2, 16) | q3
    o_ref[:, :_D] = lax.bitcast_convert_type(w01, jnp.float32)
    o_ref[:, _D:] = lax.bitcast_convert_type(w23, jnp.float32)


@jax.jit
def _transpose_pack(tt):
    return pl.pallas_call(
        _tr_body,
        grid=(_NP,),
        in_specs=[pl.BlockSpec((_D, _C), lambda i: (0, i))],
        out_specs=pl.BlockSpec((_G, 2 * _D), lambda i: (i, 0)),
        out_shape=jax.ShapeDtypeStruct((_ROWS, 2 * _D), jnp.float32),
    )(tt)


@functools.cache
def _make_sc_gather():
    mesh = plsc.VectorSubcoreMesh(core_axis_name="c", subcore_axis_name="s")

    @functools.partial(
        pl.kernel,
        mesh=mesh,
        out_type=jax.ShapeDtypeStruct((_B, 2 * _D), jnp.float32),
        scratch_types=[
            pltpu.VMEM((_BPW,), jnp.int32),
            pltpu.VMEM((_BPW, 2 * _D), jnp.float32),
            pltpu.SemaphoreType.DMA,
        ],
    )
    def _sc_gather(idx_hbm, table_hbm, out_hbm, idx_v, rows_v, sem):
        wid = lax.axis_index("s") * _NC + lax.axis_index("c")
        base = wid * _BPW
        pltpu.sync_copy(idx_hbm.at[pl.ds(base, _BPW)], idx_v)
        pltpu.async_copy(table_hbm.at[idx_v], rows_v, sem).wait()
        pltpu.sync_copy(rows_v, out_hbm.at[pl.ds(base, _BPW)])

    return _sc_gather


def _mlp_body(x_ref, sub_ref, w1_ref, b1_ref, w2_ref, b2_ref, o_ref):
    sub = sub_ref[...]
    xw = lax.bitcast_convert_type(
        jnp.where(sub < 2, x_ref[:, :_D], x_ref[:, _D:]), jnp.int32
    )
    hi = (sub % 2) == 0
    bits = jnp.where(hi, xw & jnp.int32(-65536), lax.shift_left(xw, 16))
    x = lax.bitcast_convert_type(bits, jnp.float32)
    h = jnp.dot(x, w1_ref[...], preferred_element_type=jnp.float32) + b1_ref[...]
    h = h * jax.nn.sigmoid(h)
    out = jnp.dot(h, w2_ref[...], preferred_element_type=jnp.float32) + b2_ref[...]
    o_ref[...] = out.T


_BLK = 2048


@jax.jit
def _mlp(x, sub, W1, b1, W2, b2):
    grid = (_B // _BLK,)
    return pl.pallas_call(
        _mlp_body,
        grid=grid,
        in_specs=[
            pl.BlockSpec((_BLK, 2 * _D), lambda i: (i, 0)),
            pl.BlockSpec((_BLK, 1), lambda i: (i, 0)),
            pl.BlockSpec((_D, _D), lambda i: (0, 0)),
            pl.BlockSpec((1, _D), lambda i: (0, 0)),
            pl.BlockSpec((_D, _D), lambda i: (0, 0)),
            pl.BlockSpec((1, _D), lambda i: (0, 0)),
        ],
        out_specs=pl.BlockSpec((_D, _BLK), lambda i: (0, i)),
        out_shape=jax.ShapeDtypeStruct((_D, _B), jnp.float32),
    )(x, sub, W1, b1, W2, b2)


@jax.jit
def kernel(c, emb_table, W1, b1, W2, b2):
    trm = _transpose_pack(emb_table.T)
    w4 = c % _C
    row = (c // _C) * _G + (w4 % _G)
    sub = (w4 // _G).astype(jnp.int32).reshape(_B, 1)
    gathered = _make_sc_gather()(row, trm)
    out_t = _mlp(gathered, sub, W1, b1.reshape(1, _D), W2, b2.reshape(1, _D))
    return out_t.T


# speedup vs baseline: 3.2374x; 1.1374x over previous
import functools

import jax
import jax.numpy as jnp
from jax import lax
from jax.experimental import pallas as pl
from jax.experimental.pallas import tpu as pltpu
from jax.experimental.pallas import tpu_sc as plsc

_B = 16384
_D = 64
_V = 1000000

_G = 4096
_C = 4 * _G
_NP = (_V + _C - 1) // _C
_ROWS = _NP * _G

_NC, _NS = 2, 16
_NW = _NC * _NS
_BPW = _B // _NW


def _to_bf16_bits(v):
    b = lax.bitcast_convert_type(v, jnp.int32)
    return lax.shift_right_logical(b + 0x8000, 16)


def _tr_body(x_ref, eye_ref, o_ref):
    xb = x_ref[...].astype(jnp.bfloat16)
    y = lax.dot_general(
        xb, eye_ref[...], (((0,), (0,)), ((), ())),
        preferred_element_type=jnp.float32,
    )
    q0 = _to_bf16_bits(y[:_G])
    q1 = _to_bf16_bits(y[_G : 2 * _G])
    q2 = _to_bf16_bits(y[2 * _G : 3 * _G])
    q3 = _to_bf16_bits(y[3 * _G :])
    w01 = lax.shift_left(q0, 16) | q1
    w23 = lax.shift_left(q2, 16) | q3
    o_ref[:, :_D] = lax.bitcast_convert_type(w01, jnp.float32)
    o_ref[:, _D:] = lax.bitcast_convert_type(w23, jnp.float32)


@jax.jit
def _transpose_pack(tt):
    eye = jnp.eye(_D, dtype=jnp.bfloat16)
    return pl.pallas_call(
        _tr_body,
        grid=(_NP,),
        in_specs=[
            pl.BlockSpec((_D, _C), lambda i: (0, i)),
            pl.BlockSpec((_D, _D), lambda i: (0, 0)),
        ],
        out_specs=pl.BlockSpec((_G, 2 * _D), lambda i: (i, 0)),
        out_shape=jax.ShapeDtypeStruct((_ROWS, 2 * _D), jnp.float32),
        compiler_params=pltpu.CompilerParams(fuse_transposed_lhs_in_matmul=True),
    )(tt, eye)


@functools.cache
def _make_sc_gather():
    mesh = plsc.VectorSubcoreMesh(core_axis_name="c", subcore_axis_name="s")

    @functools.partial(
        pl.kernel,
        mesh=mesh,
        out_type=jax.ShapeDtypeStruct((_B, 2 * _D), jnp.float32),
        scratch_types=[
            pltpu.VMEM((_BPW,), jnp.int32),
            pltpu.VMEM((_BPW, 2 * _D), jnp.float32),
            pltpu.SemaphoreType.DMA,
        ],
    )
    def _sc_gather(idx_hbm, table_hbm, out_hbm, idx_v, rows_v, sem):
        wid = lax.axis_index("s") * _NC + lax.axis_index("c")
        base = wid * _BPW
        pltpu.sync_copy(idx_hbm.at[pl.ds(base, _BPW)], idx_v)
        pltpu.async_copy(table_hbm.at[idx_v], rows_v, sem).wait()
        pltpu.sync_copy(rows_v, out_hbm.at[pl.ds(base, _BPW)])

    return _sc_gather


def _mlp_body(x_ref, sub_ref, w1_ref, b1_ref, w2_ref, b2_ref, o_ref):
    sub = sub_ref[...]
    xw = lax.bitcast_convert_type(
        jnp.where(sub < 2, x_ref[:, :_D], x_ref[:, _D:]), jnp.int32
    )
    hi = (sub % 2) == 0
    bits = jnp.where(hi, xw & jnp.int32(-65536), lax.shift_left(xw, 16))
    x = lax.bitcast_convert_type(bits, jnp.float32)
    h = jnp.dot(x, w1_ref[...], preferred_element_type=jnp.float32) + b1_ref[...]
    h = h * jax.nn.sigmoid(h)
    out = jnp.dot(h, w2_ref[...], preferred_element_type=jnp.float32) + b2_ref[...]
    o_ref[...] = out.T


_BLK = 2048


@jax.jit
def _mlp(x, sub, W1, b1, W2, b2):
    grid = (_B // _BLK,)
    return pl.pallas_call(
        _mlp_body,
        grid=grid,
        in_specs=[
            pl.BlockSpec((_BLK, 2 * _D), lambda i: (i, 0)),
            pl.BlockSpec((_BLK, 1), lambda i: (i, 0)),
            pl.BlockSpec((_D, _D), lambda i: (0, 0)),
            pl.BlockSpec((1, _D), lambda i: (0, 0)),
            pl.BlockSpec((_D, _D), lambda i: (0, 0)),
            pl.BlockSpec((1, _D), lambda i: (0, 0)),
        ],
        out_specs=pl.BlockSpec((_D, _BLK), lambda i: (0, i)),
        out_shape=jax.ShapeDtypeStruct((_D, _B), jnp.float32),
    )(x, sub, W1, b1, W2, b2)


@jax.jit
def kernel(c, emb_table, W1, b1, W2, b2):
    trm = _transpose_pack(emb_table.T)
    w4 = c % _C
    row = (c // _C) * _G + (w4 % _G)
    sub = (w4 // _G).astype(jnp.int32).reshape(_B, 1)
    gathered = _make_sc_gather()(row, trm)
    out_t = _mlp(gathered, sub, W1, b1.reshape(1, _D), W2, b2.reshape(1, _D))
    return out_t.T


# speedup vs baseline: 3.5747x; 1.1042x over previous
import functools

import jax
import jax.numpy as jnp
from jax import lax
from jax.experimental import pallas as pl
from jax.experimental.pallas import tpu as pltpu
from jax.experimental.pallas import tpu_sc as plsc

_B = 16384
_D = 64
_V = 1000000

_G = 8192
_C = 4 * _G
_NP = (_V + _C - 1) // _C
_ROWS = _NP * _G

_NC, _NS = 2, 16
_NW = _NC * _NS
_BPW = _B // _NW


def _to_bf16_bits(v):
    b = lax.bitcast_convert_type(v, jnp.int32)
    return lax.shift_right_logical(b + 0x8000, 16)


def _tr_body(x_ref, eye_ref, o_ref):
    xb = x_ref[...].astype(jnp.bfloat16)
    y = lax.dot_general(
        xb, eye_ref[...], (((0,), (0,)), ((), ())),
        preferred_element_type=jnp.float32,
    )
    q0 = _to_bf16_bits(y[:_G])
    q1 = _to_bf16_bits(y[_G : 2 * _G])
    q2 = _to_bf16_bits(y[2 * _G : 3 * _G])
    q3 = _to_bf16_bits(y[3 * _G :])
    w01 = lax.shift_left(q0, 16) | q1
    w23 = lax.shift_left(q2, 16) | q3
    o_ref[:, :_D] = lax.bitcast_convert_type(w01, jnp.float32)
    o_ref[:, _D:] = lax.bitcast_convert_type(w23, jnp.float32)


@jax.jit
def _transpose_pack(tt):
    eye = jnp.eye(_D, dtype=jnp.bfloat16)
    return pl.pallas_call(
        _tr_body,
        grid=(_NP,),
        in_specs=[
            pl.BlockSpec((_D, _C), lambda i: (0, i)),
            pl.BlockSpec((_D, _D), lambda i: (0, 0)),
        ],
        out_specs=pl.BlockSpec((_G, 2 * _D), lambda i: (i, 0)),
        out_shape=jax.ShapeDtypeStruct((_ROWS, 2 * _D), jnp.float32),
        compiler_params=pltpu.CompilerParams(fuse_transposed_lhs_in_matmul=True),
    )(tt, eye)


@functools.cache
def _make_sc_gather():
    mesh = plsc.VectorSubcoreMesh(core_axis_name="c", subcore_axis_name="s")

    @functools.partial(
        pl.kernel,
        mesh=mesh,
        out_type=jax.ShapeDtypeStruct((_B, 2 * _D), jnp.float32),
        scratch_types=[
            pltpu.VMEM((_BPW,), jnp.int32),
            pltpu.VMEM((_BPW, 2 * _D), jnp.float32),
            pltpu.SemaphoreType.DMA,
        ],
    )
    def _sc_gather(idx_hbm, table_hbm, out_hbm, idx_v, rows_v, sem):
        wid = lax.axis_index("s") * _NC + lax.axis_index("c")
        base = wid * _BPW
        pltpu.sync_copy(idx_hbm.at[pl.ds(base, _BPW)], idx_v)
        pltpu.async_copy(table_hbm.at[idx_v], rows_v, sem).wait()
        pltpu.sync_copy(rows_v, out_hbm.at[pl.ds(base, _BPW)])

    return _sc_gather


def _mlp_body(x_ref, sub_ref, w1_ref, b1_ref, w2_ref, b2_ref, o_ref):
    sub = sub_ref[...]
    xw = lax.bitcast_convert_type(
        jnp.where(sub < 2, x_ref[:, :_D], x_ref[:, _D:]), jnp.int32
    )
    hi = (sub % 2) == 0
    bits = jnp.where(hi, xw & jnp.int32(-65536), lax.shift_left(xw, 16))
    x = lax.bitcast_convert_type(bits, jnp.float32)
    h = jnp.dot(x, w1_ref[...], preferred_element_type=jnp.float32) + b1_ref[...]
    h = h * jax.nn.sigmoid(h)
    out = jnp.dot(h, w2_ref[...], preferred_element_type=jnp.float32) + b2_ref[...]
    o_ref[...] = out.T


_BLK = 2048


@jax.jit
def _mlp(x, sub, W1, b1, W2, b2):
    grid = (_B // _BLK,)
    return pl.pallas_call(
        _mlp_body,
        grid=grid,
        in_specs=[
            pl.BlockSpec((_BLK, 2 * _D), lambda i: (i, 0)),
            pl.BlockSpec((_BLK, 1), lambda i: (i, 0)),
            pl.BlockSpec((_D, _D), lambda i: (0, 0)),
            pl.BlockSpec((1, _D), lambda i: (0, 0)),
            pl.BlockSpec((_D, _D), lambda i: (0, 0)),
            pl.BlockSpec((1, _D), lambda i: (0, 0)),
        ],
        out_specs=pl.BlockSpec((_D, _BLK), lambda i: (0, i)),
        out_shape=jax.ShapeDtypeStruct((_D, _B), jnp.float32),
    )(x, sub, W1, b1, W2, b2)


@jax.jit
def kernel(c, emb_table, W1, b1, W2, b2):
    trm = _transpose_pack(emb_table.T)
    w4 = c % _C
    row = (c // _C) * _G + (w4 % _G)
    sub = (w4 // _G).astype(jnp.int32).reshape(_B, 1)
    gathered = _make_sc_gather()(row, trm)
    out_t = _mlp(gathered, sub, W1, b1.reshape(1, _D), W2, b2.reshape(1, _D))
    return out_t.T


# speedup vs baseline: 3.6239x; 1.0138x over previous
import functools

import jax
import jax.numpy as jnp
from jax import lax
from jax.experimental import pallas as pl
from jax.experimental.pallas import tpu as pltpu
from jax.experimental.pallas import tpu_sc as plsc

_B = 16384
_D = 64
_V = 1000000

_G = 12288
_C = 4 * _G
_NP = (_V + _C - 1) // _C
_ROWS = _NP * _G

_NC, _NS = 2, 16
_NW = _NC * _NS
_BPW = _B // _NW


def _to_bf16_bits(v):
    b = lax.bitcast_convert_type(v, jnp.int32)
    return lax.shift_right_logical(b + 0x8000, 16)


def _tr_body(x_ref, eye_ref, o_ref):
    xb = x_ref[...].astype(jnp.bfloat16)
    y = lax.dot_general(
        xb, eye_ref[...], (((0,), (0,)), ((), ())),
        preferred_element_type=jnp.float32,
    )
    q0 = _to_bf16_bits(y[:_G])
    q1 = _to_bf16_bits(y[_G : 2 * _G])
    q2 = _to_bf16_bits(y[2 * _G : 3 * _G])
    q3 = _to_bf16_bits(y[3 * _G :])
    w01 = lax.shift_left(q0, 16) | q1
    w23 = lax.shift_left(q2, 16) | q3
    o_ref[:, :_D] = lax.bitcast_convert_type(w01, jnp.float32)
    o_ref[:, _D:] = lax.bitcast_convert_type(w23, jnp.float32)


@jax.jit
def _transpose_pack(tt):
    eye = jnp.eye(_D, dtype=jnp.bfloat16)
    return pl.pallas_call(
        _tr_body,
        grid=(_NP,),
        in_specs=[
            pl.BlockSpec((_D, _C), lambda i: (0, i)),
            pl.BlockSpec((_D, _D), lambda i: (0, 0)),
        ],
        out_specs=pl.BlockSpec((_G, 2 * _D), lambda i: (i, 0)),
        out_shape=jax.ShapeDtypeStruct((_ROWS, 2 * _D), jnp.float32),
        compiler_params=pltpu.CompilerParams(fuse_transposed_lhs_in_matmul=True),
    )(tt, eye)


@functools.cache
def _make_sc_gather():
    mesh = plsc.VectorSubcoreMesh(core_axis_name="c", subcore_axis_name="s")

    @functools.partial(
        pl.kernel,
        mesh=mesh,
        out_type=jax.ShapeDtypeStruct((_B, 2 * _D), jnp.float32),
        scratch_types=[
            pltpu.VMEM((_BPW,), jnp.int32),
            pltpu.VMEM((_BPW, 2 * _D), jnp.float32),
            pltpu.SemaphoreType.DMA,
        ],
    )
    def _sc_gather(idx_hbm, table_hbm, out_hbm, idx_v, rows_v, sem):
        wid = lax.axis_index("s") * _NC + lax.axis_index("c")
        base = wid * _BPW
        pltpu.sync_copy(idx_hbm.at[pl.ds(base, _BPW)], idx_v)
        pltpu.async_copy(table_hbm.at[idx_v], rows_v, sem).wait()
        pltpu.sync_copy(rows_v, out_hbm.at[pl.ds(base, _BPW)])

    return _sc_gather


def _mlp_body(x_ref, sub_ref, w1_ref, b1_ref, w2_ref, b2_ref, o_ref):
    sub = sub_ref[...]
    xw = lax.bitcast_convert_type(
        jnp.where(sub < 2, x_ref[:, :_D], x_ref[:, _D:]), jnp.int32
    )
    hi = (sub % 2) == 0
    bits = jnp.where(hi, xw & jnp.int32(-65536), lax.shift_left(xw, 16))
    x = lax.bitcast_convert_type(bits, jnp.float32)
    h = jnp.dot(x, w1_ref[...], preferred_element_type=jnp.float32) + b1_ref[...]
    h = h * jax.nn.sigmoid(h)
    out = jnp.dot(h, w2_ref[...], preferred_element_type=jnp.float32) + b2_ref[...]
    o_ref[...] = out.T


_BLK = 2048


@jax.jit
def _mlp(x, sub, W1, b1, W2, b2):
    grid = (_B // _BLK,)
    return pl.pallas_call(
        _mlp_body,
        grid=grid,
        in_specs=[
            pl.BlockSpec((_BLK, 2 * _D), lambda i: (i, 0)),
            pl.BlockSpec((_BLK, 1), lambda i: (i, 0)),
            pl.BlockSpec((_D, _D), lambda i: (0, 0)),
            pl.BlockSpec((1, _D), lambda i: (0, 0)),
            pl.BlockSpec((_D, _D), lambda i: (0, 0)),
            pl.BlockSpec((1, _D), lambda i: (0, 0)),
        ],
        out_specs=pl.BlockSpec((_D, _BLK), lambda i: (0, i)),
        out_shape=jax.ShapeDtypeStruct((_D, _B), jnp.float32),
    )(x, sub, W1, b1, W2, b2)


@jax.jit
def kernel(c, emb_table, W1, b1, W2, b2):
    trm = _transpose_pack(emb_table.T)
    w4 = c % _C
    row = (c // _C) * _G + (w4 % _G)
    sub = (w4 // _G).astype(jnp.int32).reshape(_B, 1)
    gathered = _make_sc_gather()(row, trm)
    out_t = _mlp(gathered, sub, W1, b1.reshape(1, _D), W2, b2.reshape(1, _D))
    return out_t.T


# speedup vs baseline: 3.6660x; 1.0116x over previous
import functools

import jax
import jax.numpy as jnp
from jax import lax
from jax.experimental import pallas as pl
from jax.experimental.pallas import tpu as pltpu
from jax.experimental.pallas import tpu_sc as plsc

_B = 16384
_D = 64
_V = 1000000

_G = 14336
_C = 4 * _G
_NP = (_V + _C - 1) // _C
_ROWS = _NP * _G

_NC, _NS = 2, 16
_NW = _NC * _NS
_BPW = _B // _NW


def _to_bf16_bits(v):
    b = lax.bitcast_convert_type(v, jnp.int32)
    return lax.shift_right_logical(b + 0x8000, 16)


def _tr_body(x_ref, eye_ref, o_ref):
    xb = x_ref[...].astype(jnp.bfloat16)
    y = lax.dot_general(
        xb, eye_ref[...], (((0,), (0,)), ((), ())),
        preferred_element_type=jnp.float32,
    )
    q0 = _to_bf16_bits(y[:_G])
    q1 = _to_bf16_bits(y[_G : 2 * _G])
    q2 = _to_bf16_bits(y[2 * _G : 3 * _G])
    q3 = _to_bf16_bits(y[3 * _G :])
    w01 = lax.shift_left(q0, 16) | q1
    w23 = lax.shift_left(q2, 16) | q3
    o_ref[:, :_D] = lax.bitcast_convert_type(w01, jnp.float32)
    o_ref[:, _D:] = lax.bitcast_convert_type(w23, jnp.float32)


@jax.jit
def _transpose_pack(tt):
    eye = jnp.eye(_D, dtype=jnp.bfloat16)
    return pl.pallas_call(
        _tr_body,
        grid=(_NP,),
        in_specs=[
            pl.BlockSpec((_D, _C), lambda i: (0, i)),
            pl.BlockSpec((_D, _D), lambda i: (0, 0)),
        ],
        out_specs=pl.BlockSpec((_G, 2 * _D), lambda i: (i, 0)),
        out_shape=jax.ShapeDtypeStruct((_ROWS, 2 * _D), jnp.float32),
        compiler_params=pltpu.CompilerParams(fuse_transposed_lhs_in_matmul=True),
    )(tt, eye)


@functools.cache
def _make_sc_gather():
    mesh = plsc.VectorSubcoreMesh(core_axis_name="c", subcore_axis_name="s")

    @functools.partial(
        pl.kernel,
        mesh=mesh,
        out_type=jax.ShapeDtypeStruct((_B, 2 * _D), jnp.float32),
        scratch_types=[
            pltpu.VMEM((_BPW,), jnp.int32),
            pltpu.VMEM((_BPW, 2 * _D), jnp.float32),
            pltpu.SemaphoreType.DMA,
        ],
    )
    def _sc_gather(idx_hbm, table_hbm, out_hbm, idx_v, rows_v, sem):
        wid = lax.axis_index("s") * _NC + lax.axis_index("c")
        base = wid * _BPW
        pltpu.sync_copy(idx_hbm.at[pl.ds(base, _BPW)], idx_v)
        pltpu.async_copy(table_hbm.at[idx_v], rows_v, sem).wait()
        pltpu.sync_copy(rows_v, out_hbm.at[pl.ds(base, _BPW)])

    return _sc_gather


def _mlp_body(x_ref, sub_ref, w1_ref, b1_ref, w2_ref, b2_ref, o_ref):
    sub = sub_ref[...]
    xw = lax.bitcast_convert_type(
        jnp.where(sub < 2, x_ref[:, :_D], x_ref[:, _D:]), jnp.int32
    )
    hi = (sub % 2) == 0
    bits = jnp.where(hi, xw & jnp.int32(-65536), lax.shift_left(xw, 16))
    x = lax.bitcast_convert_type(bits, jnp.float32)
    h = jnp.dot(x, w1_ref[...], preferred_element_type=jnp.float32) + b1_ref[...]
    h = h * jax.nn.sigmoid(h)
    out = jnp.dot(h, w2_ref[...], preferred_element_type=jnp.float32) + b2_ref[...]
    o_ref[...] = out.T


_BLK = 2048


@jax.jit
def _mlp(x, sub, W1, b1, W2, b2):
    grid = (_B // _BLK,)
    return pl.pallas_call(
        _mlp_body,
        grid=grid,
        in_specs=[
            pl.BlockSpec((_BLK, 2 * _D), lambda i: (i, 0)),
            pl.BlockSpec((_BLK, 1), lambda i: (i, 0)),
            pl.BlockSpec((_D, _D), lambda i: (0, 0)),
            pl.BlockSpec((1, _D), lambda i: (0, 0)),
            pl.BlockSpec((_D, _D), lambda i: (0, 0)),
            pl.BlockSpec((1, _D), lambda i: (0, 0)),
        ],
        out_specs=pl.BlockSpec((_D, _BLK), lambda i: (0, i)),
        out_shape=jax.ShapeDtypeStruct((_D, _B), jnp.float32),
    )(x, sub, W1, b1, W2, b2)


@jax.jit
def kernel(c, emb_table, W1, b1, W2, b2):
    trm = _transpose_pack(emb_table.T)
    w4 = c % _C
    row = (c // _C) * _G + (w4 % _G)
    sub = (w4 // _G).astype(jnp.int32).reshape(_B, 1)
    gathered = _make_sc_gather()(row, trm)
    out_t = _mlp(gathered, sub, W1, b1.reshape(1, _D), W2, b2.reshape(1, _D))
    return out_t.T


# speedup vs baseline: 3.6673x; 1.0004x over previous
import functools

import jax
import jax.numpy as jnp
from jax import lax
from jax.experimental import pallas as pl
from jax.experimental.pallas import tpu as pltpu
from jax.experimental.pallas import tpu_sc as plsc

_B = 16384
_D = 64
_V = 1000000

_G = 14336
_C = 4 * _G
_NP = (_V + _C - 1) // _C
_ROWS = _NP * _G

_NC, _NS = 2, 16
_NW = _NC * _NS
_BPW = _B // _NW


def _to_bf16_bits(v):
    b = lax.bitcast_convert_type(v, jnp.int32)
    return lax.shift_right_logical(b + 0x8000, 16)


def _tr_body(x_ref, eye_ref, o_ref):
    xb = x_ref[...].astype(jnp.bfloat16)
    y = lax.dot_general(
        xb, eye_ref[...], (((0,), (0,)), ((), ())),
        preferred_element_type=jnp.float32,
    )
    q0 = _to_bf16_bits(y[:_G])
    q1 = _to_bf16_bits(y[_G : 2 * _G])
    q2 = _to_bf16_bits(y[2 * _G : 3 * _G])
    q3 = _to_bf16_bits(y[3 * _G :])
    w01 = lax.shift_left(q0, 16) | q1
    w23 = lax.shift_left(q2, 16) | q3
    o_ref[:, :_D] = lax.bitcast_convert_type(w01, jnp.float32)
    o_ref[:, _D:] = lax.bitcast_convert_type(w23, jnp.float32)


@jax.jit
def _transpose_pack(tt):
    eye = jnp.eye(_D, dtype=jnp.bfloat16)
    return pl.pallas_call(
        _tr_body,
        grid=(_NP,),
        in_specs=[
            pl.BlockSpec((_D, _C), lambda i: (0, i)),
            pl.BlockSpec((_D, _D), lambda i: (0, 0)),
        ],
        out_specs=pl.BlockSpec((_G, 2 * _D), lambda i: (i, 0)),
        out_shape=jax.ShapeDtypeStruct((_ROWS, 2 * _D), jnp.float32),
        compiler_params=pltpu.CompilerParams(fuse_transposed_lhs_in_matmul=True),
    )(tt, eye)


@functools.cache
def _make_sc_gather():
    mesh = plsc.VectorSubcoreMesh(core_axis_name="c", subcore_axis_name="s")

    @functools.partial(
        pl.kernel,
        mesh=mesh,
        out_type=jax.ShapeDtypeStruct((_B, 2 * _D), jnp.float32),
        scratch_types=[
            pltpu.VMEM((_BPW,), jnp.int32),
            pltpu.VMEM((_BPW, 2 * _D), jnp.float32),
            pltpu.SemaphoreType.DMA,
        ],
    )
    def _sc_gather(idx_hbm, table_hbm, out_hbm, idx_v, rows_v, sem):
        wid = lax.axis_index("s") * _NC + lax.axis_index("c")
        base = wid * _BPW
        pltpu.sync_copy(idx_hbm.at[pl.ds(base, _BPW)], idx_v)
        pltpu.async_copy(table_hbm.at[idx_v], rows_v, sem).wait()
        pltpu.sync_copy(rows_v, out_hbm.at[pl.ds(base, _BPW)])

    return _sc_gather


def _mlp_body(x_ref, sub_ref, w1_ref, b1_ref, w2_ref, b2_ref, o_ref):
    sub = sub_ref[...]
    xw = lax.bitcast_convert_type(
        jnp.where(sub < 2, x_ref[:, :_D], x_ref[:, _D:]), jnp.int32
    )
    hi = (sub % 2) == 0
    bits = jnp.where(hi, xw & jnp.int32(-65536), lax.shift_left(xw, 16))
    x = lax.bitcast_convert_type(bits, jnp.float32)
    h = jnp.dot(x, w1_ref[...], preferred_element_type=jnp.float32) + b1_ref[...]
    h = h * jax.nn.sigmoid(h)
    out = jnp.dot(h, w2_ref[...], preferred_element_type=jnp.float32) + b2_ref[...]
    o_ref[...] = out.T


_BLK = 4096


@jax.jit
def _mlp(x, sub, W1, b1, W2, b2):
    grid = (_B // _BLK,)
    return pl.pallas_call(
        _mlp_body,
        grid=grid,
        in_specs=[
            pl.BlockSpec((_BLK, 2 * _D), lambda i: (i, 0)),
            pl.BlockSpec((_BLK, 1), lambda i: (i, 0)),
            pl.BlockSpec((_D, _D), lambda i: (0, 0)),
            pl.BlockSpec((1, _D), lambda i: (0, 0)),
            pl.BlockSpec((_D, _D), lambda i: (0, 0)),
            pl.BlockSpec((1, _D), lambda i: (0, 0)),
        ],
        out_specs=pl.BlockSpec((_D, _BLK), lambda i: (0, i)),
        out_shape=jax.ShapeDtypeStruct((_D, _B), jnp.float32),
    )(x, sub, W1, b1, W2, b2)


@jax.jit
def kernel(c, emb_table, W1, b1, W2, b2):
    trm = _transpose_pack(emb_table.T)
    w4 = c % _C
    row = (c // _C) * _G + (w4 % _G)
    sub = (w4 // _G).astype(jnp.int32).reshape(_B, 1)
    gathered = _make_sc_gather()(row, trm)
    out_t = _mlp(gathered, sub, W1, b1.reshape(1, _D), W2, b2.reshape(1, _D))
    return out_t.T
